# baseline (speedup 1.0000x reference)
import functools

import jax
import jax.numpy as jnp
from jax import lax
from jax.experimental import pallas as pl
from jax.experimental.pallas import tpu as pltpu
from jax.experimental.pallas import tpu_sc as plsc


@functools.lru_cache(maxsize=None)
def _build(n: int, d: int):
    info = plsc.get_sparse_core_info()
    nc, ns = 1, info.num_subcores
    nw = nc * ns
    assert n % nw == 0
    rows_per = n // nw
    mesh = plsc.VectorSubcoreMesh(
        core_axis_name="c", subcore_axis_name="s", num_cores=1
    )

    half = rows_per // 2

    @functools.partial(
        pl.kernel,
        mesh=mesh,
        out_type=jax.ShapeDtypeStruct((n, d), jnp.float32),
        scratch_types=[
            pltpu.VMEM((half, d), jnp.float32),
            pltpu.VMEM((half, d), jnp.float32),
            pltpu.SemaphoreType.DMA,
            pltpu.SemaphoreType.DMA,
        ],
    )
    def body(w_hbm, out_hbm, v0, v1, s0, s1):
        wid = lax.axis_index("s") * nc + lax.axis_index("c")
        base = wid * rows_per
        g0 = pltpu.async_copy(w_hbm.at[pl.ds(base, half)], v0, s0)
        g1 = pltpu.async_copy(w_hbm.at[pl.ds(base + half, half)], v1, s1)
        g0.wait()
        p0 = pltpu.async_copy(v0, out_hbm.at[pl.ds(base, half)], s0)
        g1.wait()
        p1 = pltpu.async_copy(v1, out_hbm.at[pl.ds(base + half, half)], s1)
        p0.wait()
        p1.wait()

    return body


def kernel(input, weights):
    n = input.shape[0]
    d = weights.shape[1]
    return _build(n, d)(weights)


# speedup vs baseline: 1.0006x; 1.0006x over previous
import functools

import jax
import jax.numpy as jnp
from jax import lax
from jax.experimental import pallas as pl
from jax.experimental.pallas import tpu as pltpu
from jax.experimental.pallas import tpu_sc as plsc


@functools.lru_cache(maxsize=None)
def _build(n: int, d: int):
    info = plsc.get_sparse_core_info()
    nc, ns = 1, info.num_subcores
    nw = nc * ns
    assert n % (2 * nw) == 0
    rows_per = n // nw
    mesh = plsc.VectorSubcoreMesh(
        core_axis_name="c", subcore_axis_name="s", num_cores=1
    )

    half = rows_per // 2

    @functools.partial(
        pl.kernel,
        mesh=mesh,
        out_type=jax.ShapeDtypeStruct((n, d), jnp.float32),
        scratch_types=[
            pltpu.VMEM((half, d), jnp.float32),
            pltpu.VMEM((half, d), jnp.float32),
            pltpu.SemaphoreType.DMA,
            pltpu.SemaphoreType.DMA,
        ],
    )
    def body(w_hbm, out_hbm, v0, v1, s0, s1):
        wid = lax.axis_index("s") * nc + lax.axis_index("c")
        base = wid * rows_per
        g0 = pltpu.async_copy(w_hbm.at[pl.ds(base, half)], v0, s0)
        g1 = pltpu.async_copy(w_hbm.at[pl.ds(base + half, half)], v1, s1)
        g0.wait()
        p0 = pltpu.async_copy(v0, out_hbm.at[pl.ds(base, half)], s0)
        g1.wait()
        p1 = pltpu.async_copy(v1, out_hbm.at[pl.ds(base + half, half)], s1)
        p0.wait()
        p1.wait()

    return body


def kernel(input, weights):
    n = input.shape[0]
    d = weights.shape[1]
    return _build(n, d)(weights)


# speedup vs baseline: 1.0046x; 1.0039x over previous
import functools

import jax
import jax.numpy as jnp
from jax import lax
from jax.experimental import pallas as pl
from jax.experimental.pallas import tpu as pltpu
from jax.experimental.pallas import tpu_sc as plsc


@functools.lru_cache(maxsize=None)
def _build(n: int, d: int):
    info = plsc.get_sparse_core_info()
    nc, ns = 1, info.num_subcores
    nw = nc * ns
    assert n % (2 * nw) == 0
    rows_per = n // nw
    mesh = plsc.VectorSubcoreMesh(
        core_axis_name="c", subcore_axis_name="s", num_cores=1
    )

    half = rows_per // 2

    @functools.partial(
        pl.kernel,
        mesh=mesh,
        out_type=jax.ShapeDtypeStruct((n, d), jnp.float32),
        compiler_params=pltpu.CompilerParams(allow_input_fusion=[True]),
        scratch_types=[
            pltpu.VMEM((half, d), jnp.float32),
            pltpu.VMEM((half, d), jnp.float32),
            pltpu.SemaphoreType.DMA,
            pltpu.SemaphoreType.DMA,
        ],
    )
    def body(w_hbm, out_hbm, v0, v1, s0, s1):
        wid = lax.axis_index("s") * nc + lax.axis_index("c")
        base = wid * rows_per
        g0 = pltpu.async_copy(w_hbm.at[pl.ds(base, half)], v0, s0)
        g1 = pltpu.async_copy(w_hbm.at[pl.ds(base + half, half)], v1, s1)
        g0.wait()
        p0 = pltpu.async_copy(v0, out_hbm.at[pl.ds(base, half)], s0)
        g1.wait()
        p1 = pltpu.async_copy(v1, out_hbm.at[pl.ds(base + half, half)], s1)
        p0.wait()
        p1.wait()

    return body


def kernel(input, weights):
    n = input.shape[0]
    d = weights.shape[1]
    return _build(n, d)(weights)


# speedup vs baseline: 1.0410x; 1.0362x over previous
import functools

import jax
import jax.numpy as jnp
from jax import lax
from jax.experimental import pallas as pl
from jax.experimental.pallas import tpu as pltpu
from jax.experimental.pallas import tpu_sc as plsc


@functools.lru_cache(maxsize=None)
def _build(total: int):
    info = plsc.get_sparse_core_info()
    nc, ns = 1, info.num_subcores
    nw = nc * ns
    assert total % (2 * nw) == 0
    per = total // nw
    half = per // 2
    mesh = plsc.VectorSubcoreMesh(
        core_axis_name="c", subcore_axis_name="s", num_cores=1
    )

    @functools.partial(
        pl.kernel,
        mesh=mesh,
        out_type=jax.ShapeDtypeStruct((total,), jnp.float32),
        scratch_types=[
            pltpu.VMEM((half,), jnp.float32),
            pltpu.VMEM((half,), jnp.float32),
            pltpu.SemaphoreType.DMA,
            pltpu.SemaphoreType.DMA,
        ],
    )
    def body(w_hbm, out_hbm, v0, v1, s0, s1):
        wid = lax.axis_index("s") * nc + lax.axis_index("c")
        base = wid * per
        g0 = pltpu.async_copy(w_hbm.at[pl.ds(base, half)], v0, s0)
        g1 = pltpu.async_copy(w_hbm.at[pl.ds(base + half, half)], v1, s1)
        g0.wait()
        p0 = pltpu.async_copy(v0, out_hbm.at[pl.ds(base, half)], s0)
        g1.wait()
        p1 = pltpu.async_copy(v1, out_hbm.at[pl.ds(base + half, half)], s1)
        p0.wait()
        p1.wait()

    return body


def kernel(input, weights):
    n = input.shape[0]
    d = weights.shape[1]
    flat = jnp.reshape(weights, (-1,))
    out = _build(n * d)(flat)
    return jnp.reshape(out, (n, d))
